# baseline (speedup 1.0000x reference)
import functools

import jax
import jax.numpy as jnp
from jax.experimental import pallas as pl

INPUT_DIM = 768
EMBED_DIM = 64
K = 1024
L = 3
N = 20

_DN = (((1,), (1,)), ((), ()))


def _bf16_dot(a, b):
    return jnp.dot(a.astype(jnp.bfloat16), b.astype(jnp.bfloat16),
                   preferred_element_type=jnp.float32)


def _argmax_lanes(scores_neg, lane_iota):
    m = jnp.max(scores_neg, axis=-1, keepdims=True)
    return jnp.min(jnp.where(scores_neg >= m, lane_iota, K), axis=-1)


def _gather_rows(onehot, hi, mid, lo):
    g = jnp.dot(onehot, hi, preferred_element_type=jnp.float32)
    g = g + jnp.dot(onehot, mid, preferred_element_type=jnp.float32)
    return g + jnp.dot(onehot, lo, preferred_element_type=jnp.float32)


def _encode(xc, w0, b0, w1, b1, w2, b2, w3, b3):
    h = jnp.maximum(_bf16_dot(xc, w0[...]) + b0[...], 0.0)
    h = jnp.maximum(_bf16_dot(h, w1[...]) + b1[...], 0.0)
    h = jnp.maximum(_bf16_dot(h, w2[...]) + b2[...], 0.0)
    return _bf16_dot(h, w3[...]) + b3[...]


def _fused_body(x_ref, w0, b0, w1, b1, w2, b2, w3, b3,
                cb_ref, cb2_ref, hi_ref, mid_ref, lo_ref, out_ref):
    hiprec = jax.lax.Precision.HIGHEST
    bb = x_ref.shape[0]
    chains = 2
    nh = N // chains
    res = []
    for s in range(chains):
        zs = [_encode(x_ref[:, s * nh + j, :], w0, b0, w1, b1, w2, b2, w3, b3)
              for j in range(nh)]
        res.append(jnp.concatenate(zs, axis=0))

    t = bb * nh
    lane_iota = jax.lax.broadcasted_iota(jnp.int32, (t, K), 1)
    ones_col = jnp.ones((1, EMBED_DIM), jnp.float32)
    neg_ones = jnp.full((t, L), -1.0, jnp.bfloat16)
    for l in range(L):
        cb = cb_ref[l]
        cb2 = cb2_ref[l]
        cn = jax.lax.dot_general(cb * cb, ones_col, _DN, precision=hiprec,
                                 preferred_element_type=jnp.float32)
        cn1 = cn.astype(jnp.bfloat16)
        r1 = cn - cn1.astype(jnp.float32)
        cn2 = r1.astype(jnp.bfloat16)
        cn3 = (r1 - cn2.astype(jnp.float32)).astype(jnp.bfloat16)
        cb_aug = jnp.concatenate([cb2, cn1, cn2, cn3], axis=1)
        for s in range(chains):
            res_aug = jnp.concatenate(
                [res[s].astype(jnp.bfloat16), neg_ones], axis=1)
            scores_neg = jax.lax.dot_general(
                res_aug, cb_aug, _DN, preferred_element_type=jnp.float32)
            idx = _argmax_lanes(scores_neg, lane_iota)
            idx_col = idx[:, None]
            for j in range(nh):
                col = (s * nh + j) * L + l
                out_ref[:, col:col + 1] = idx_col[j * bb:(j + 1) * bb]
            if l + 1 < L:
                onehot = (lane_iota == idx_col).astype(jnp.bfloat16)
                res[s] = res[s] - _gather_rows(onehot, hi_ref[l], mid_ref[l],
                                               lo_ref[l])


@functools.partial(jax.jit, static_argnames=("block_b",))
def _run(x, W0, b0, W1, b1, W2, b2, W3, b3, codebooks, block_b=128):
    b = x.shape[0]
    grid = (b // block_b,)
    cb_hi = codebooks.astype(jnp.bfloat16)
    r1 = codebooks - cb_hi.astype(jnp.float32)
    cb_mid = r1.astype(jnp.bfloat16)
    cb_lo = (r1 - cb_mid.astype(jnp.float32)).astype(jnp.bfloat16)
    cb2 = (2.0 * codebooks).astype(jnp.bfloat16)
    full = lambda shape: pl.BlockSpec(shape, lambda i: (0,) * len(shape))
    out = pl.pallas_call(
        _fused_body,
        grid=grid,
        in_specs=[
            pl.BlockSpec((block_b, N, INPUT_DIM), lambda i: (i, 0, 0)),
            full(W0.shape), full(b0.shape),
            full(W1.shape), full(b1.shape),
            full(W2.shape), full(b2.shape),
            full(W3.shape), full(b3.shape),
            full(codebooks.shape), full(cb2.shape),
            full(cb_hi.shape), full(cb_mid.shape), full(cb_lo.shape),
        ],
        out_specs=pl.BlockSpec((block_b, N * L), lambda i: (i, 0)),
        out_shape=jax.ShapeDtypeStruct((b, N * L), jnp.int32),
    )(x, W0, b0, W1, b1, W2, b2, W3, b3,
      codebooks, cb2, cb_hi, cb_mid, cb_lo)
    return out


def kernel(user_ids, ids, ids_fut, x, x_fut, seq_mask,
           W0, b0, W1, b1, W2, b2, W3, b3, codebooks):
    b, n, _ = x.shape
    sem_ids_flat = _run(x, W0, b0, W1, b1, W2, b2, W3, b3, codebooks)
    token_type_ids = jnp.tile(jnp.arange(L, dtype=jnp.int32), (b, n))
    return sem_ids_flat, token_type_ids, user_ids


# speedup vs baseline: 1.4183x; 1.4183x over previous
import functools

import jax
import jax.numpy as jnp
from jax.experimental import pallas as pl

INPUT_DIM = 768
EMBED_DIM = 64
K = 1024
L = 3
N = 20

_DN = (((1,), (1,)), ((), ()))


def _bf16_dot(a, b):
    return jnp.dot(a.astype(jnp.bfloat16), b.astype(jnp.bfloat16),
                   preferred_element_type=jnp.float32)


def _argmax_lanes(scores_neg, lane_iota):
    del lane_iota
    return jnp.argmax(scores_neg, axis=-1).astype(jnp.int32)


def _gather_rows(onehot, hi, mid, lo):
    g = jnp.dot(onehot, hi, preferred_element_type=jnp.float32)
    g = g + jnp.dot(onehot, mid, preferred_element_type=jnp.float32)
    return g + jnp.dot(onehot, lo, preferred_element_type=jnp.float32)


def _encode(xc, w0, b0, w1, b1, w2, b2, w3, b3):
    h = jnp.maximum(_bf16_dot(xc, w0[...]) + b0[...], 0.0)
    h = jnp.maximum(_bf16_dot(h, w1[...]) + b1[...], 0.0)
    h = jnp.maximum(_bf16_dot(h, w2[...]) + b2[...], 0.0)
    return _bf16_dot(h, w3[...]) + b3[...]


def _fused_body(x_ref, w0, b0, w1, b1, w2, b2, w3, b3,
                cb_ref, cb2_ref, hi_ref, mid_ref, lo_ref, out_ref):
    hiprec = jax.lax.Precision.HIGHEST
    bb = x_ref.shape[0]
    chains = 2
    nh = N // chains
    res = []
    for s in range(chains):
        xc = jnp.concatenate(
            [x_ref[:, s * nh + j, :] for j in range(nh)], axis=0)
        res.append(_encode(xc, w0, b0, w1, b1, w2, b2, w3, b3))

    t = bb * nh
    lane_iota = jax.lax.broadcasted_iota(jnp.int32, (t, K), 1)
    ones_col = jnp.ones((1, EMBED_DIM), jnp.float32)
    neg_ones = jnp.full((t, L), -1.0, jnp.bfloat16)
    for l in range(L):
        cb = cb_ref[l]
        cb2 = cb2_ref[l]
        cn = jax.lax.dot_general(cb * cb, ones_col, _DN, precision=hiprec,
                                 preferred_element_type=jnp.float32)
        cn1 = cn.astype(jnp.bfloat16)
        r1 = cn - cn1.astype(jnp.float32)
        cn2 = r1.astype(jnp.bfloat16)
        cn3 = (r1 - cn2.astype(jnp.float32)).astype(jnp.bfloat16)
        cb_aug = jnp.concatenate([cb2, cn1, cn2, cn3], axis=1)
        for s in range(chains):
            res_aug = jnp.concatenate(
                [res[s].astype(jnp.bfloat16), neg_ones], axis=1)
            scores_neg = jax.lax.dot_general(
                res_aug, cb_aug, _DN, preferred_element_type=jnp.float32)
            idx = _argmax_lanes(scores_neg, lane_iota)
            idx_col = idx[:, None]
            for j in range(nh):
                col = (s * nh + j) * L + l
                out_ref[:, col:col + 1] = idx_col[j * bb:(j + 1) * bb]
            if l + 1 < L:
                onehot = (lane_iota == idx_col).astype(jnp.bfloat16)
                res[s] = res[s] - _gather_rows(onehot, hi_ref[l], mid_ref[l],
                                               lo_ref[l])


@functools.partial(jax.jit, static_argnames=("block_b",))
def _run(x, W0, b0, W1, b1, W2, b2, W3, b3, codebooks, block_b=128):
    b = x.shape[0]
    grid = (b // block_b,)
    cb_hi = codebooks.astype(jnp.bfloat16)
    r1 = codebooks - cb_hi.astype(jnp.float32)
    cb_mid = r1.astype(jnp.bfloat16)
    cb_lo = (r1 - cb_mid.astype(jnp.float32)).astype(jnp.bfloat16)
    cb2 = (2.0 * codebooks).astype(jnp.bfloat16)
    full = lambda shape: pl.BlockSpec(shape, lambda i: (0,) * len(shape))
    out = pl.pallas_call(
        _fused_body,
        grid=grid,
        in_specs=[
            pl.BlockSpec((block_b, N, INPUT_DIM), lambda i: (i, 0, 0)),
            full(W0.shape), full(b0.shape),
            full(W1.shape), full(b1.shape),
            full(W2.shape), full(b2.shape),
            full(W3.shape), full(b3.shape),
            full(codebooks.shape), full(cb2.shape),
            full(cb_hi.shape), full(cb_mid.shape), full(cb_lo.shape),
        ],
        out_specs=pl.BlockSpec((block_b, N * L), lambda i: (i, 0)),
        out_shape=jax.ShapeDtypeStruct((b, N * L), jnp.int32),
    )(x, W0, b0, W1, b1, W2, b2, W3, b3,
      codebooks, cb2, cb_hi, cb_mid, cb_lo)
    return out


def kernel(user_ids, ids, ids_fut, x, x_fut, seq_mask,
           W0, b0, W1, b1, W2, b2, W3, b3, codebooks):
    b, n, _ = x.shape
    sem_ids_flat = _run(x, W0, b0, W1, b1, W2, b2, W3, b3, codebooks)
    token_type_ids = jnp.tile(jnp.arange(L, dtype=jnp.int32), (b, n))
    return sem_ids_flat, token_type_ids, user_ids


# speedup vs baseline: 1.4203x; 1.0014x over previous
import functools

import jax
import jax.numpy as jnp
from jax.experimental import pallas as pl

INPUT_DIM = 768
EMBED_DIM = 64
K = 1024
L = 3
N = 20

_DN = (((1,), (1,)), ((), ()))


def _bf16_dot(a, b):
    return jnp.dot(a.astype(jnp.bfloat16), b.astype(jnp.bfloat16),
                   preferred_element_type=jnp.float32)


def _argmax_lanes(scores_neg):
    return jnp.argmax(scores_neg, axis=-1).astype(jnp.int32)


def _gather_rows(onehot, hi, mid, lo):
    g = jnp.dot(onehot, hi, preferred_element_type=jnp.float32)
    g = g + jnp.dot(onehot, mid, preferred_element_type=jnp.float32)
    return g + jnp.dot(onehot, lo, preferred_element_type=jnp.float32)


def _encode(xc, w0, b0, w1, b1, w2, b2, w3, b3):
    h = jnp.maximum(_bf16_dot(xc, w0[...]) + b0[...], 0.0)
    h = jnp.maximum(_bf16_dot(h, w1[...]) + b1[...], 0.0)
    h = jnp.maximum(_bf16_dot(h, w2[...]) + b2[...], 0.0)
    return _bf16_dot(h, w3[...]) + b3[...]


def _fused_body(x_ref, w0, b0, w1, b1, w2, b2, w3, b3,
                cb_ref, cb2_ref, hi_ref, mid_ref, lo_ref, out_ref):
    hiprec = jax.lax.Precision.HIGHEST
    bb = x_ref.shape[0]
    chains = 2
    nh = N // chains
    res = []
    for s in range(chains):
        xc = jnp.concatenate(
            [x_ref[:, s * nh + j, :] for j in range(nh)], axis=0)
        res.append(_encode(xc, w0, b0, w1, b1, w2, b2, w3, b3))

    t = bb * nh
    lane_iota = jax.lax.broadcasted_iota(jnp.int32, (t, K), 1)
    ones_col = jnp.ones((1, EMBED_DIM), jnp.float32)
    neg_ones = jnp.full((t, L), -1.0, jnp.bfloat16)
    for l in range(L):
        cb = cb_ref[l]
        cb2 = cb2_ref[l]
        cn = jax.lax.dot_general(cb * cb, ones_col, _DN, precision=hiprec,
                                 preferred_element_type=jnp.float32)
        cn1 = cn.astype(jnp.bfloat16)
        r1 = cn - cn1.astype(jnp.float32)
        cn2 = r1.astype(jnp.bfloat16)
        cn3 = (r1 - cn2.astype(jnp.float32)).astype(jnp.bfloat16)
        cb_aug = jnp.concatenate([cb2, cn1, cn2, cn3], axis=1)
        for s in range(chains):
            res_aug = jnp.concatenate(
                [res[s].astype(jnp.bfloat16), neg_ones], axis=1)
            scores_neg = jax.lax.dot_general(
                res_aug, cb_aug, _DN, preferred_element_type=jnp.float32)
            idx = _argmax_lanes(scores_neg)
            idx_col = idx[:, None]
            for j in range(nh):
                col = (s * nh + j) * L + l
                out_ref[:, col:col + 1] = idx_col[j * bb:(j + 1) * bb]
            if l + 1 < L:
                onehot = (lane_iota == idx_col).astype(jnp.bfloat16)
                res[s] = res[s] - _gather_rows(onehot, hi_ref[l], mid_ref[l],
                                               lo_ref[l])


@functools.partial(jax.jit, static_argnames=("block_b",))
def _run(x, W0, b0, W1, b1, W2, b2, W3, b3, codebooks, block_b=128):
    b = x.shape[0]
    grid = (b // block_b,)
    cb_hi = codebooks.astype(jnp.bfloat16)
    r1 = codebooks - cb_hi.astype(jnp.float32)
    cb_mid = r1.astype(jnp.bfloat16)
    cb_lo = (r1 - cb_mid.astype(jnp.float32)).astype(jnp.bfloat16)
    cb2 = (2.0 * codebooks).astype(jnp.bfloat16)
    full = lambda shape: pl.BlockSpec(shape, lambda i: (0,) * len(shape))
    out = pl.pallas_call(
        _fused_body,
        grid=grid,
        in_specs=[
            pl.BlockSpec((block_b, N, INPUT_DIM), lambda i: (i, 0, 0)),
            full(W0.shape), full(b0.shape),
            full(W1.shape), full(b1.shape),
            full(W2.shape), full(b2.shape),
            full(W3.shape), full(b3.shape),
            full(codebooks.shape), full(cb2.shape),
            full(cb_hi.shape), full(cb_mid.shape), full(cb_lo.shape),
        ],
        out_specs=pl.BlockSpec((block_b, N * L), lambda i: (i, 0)),
        out_shape=jax.ShapeDtypeStruct((b, N * L), jnp.int32),
    )(x, W0, b0, W1, b1, W2, b2, W3, b3,
      codebooks, cb2, cb_hi, cb_mid, cb_lo)
    return out


def kernel(user_ids, ids, ids_fut, x, x_fut, seq_mask,
           W0, b0, W1, b1, W2, b2, W3, b3, codebooks):
    b, n, _ = x.shape
    sem_ids_flat = _run(x, W0, b0, W1, b1, W2, b2, W3, b3, codebooks)
    token_type_ids = jnp.tile(jnp.arange(L, dtype=jnp.int32), (b, n))
    return sem_ids_flat, token_type_ids, user_ids
